# baseline (speedup 1.0000x reference)
import functools

import jax
import jax.numpy as jnp
from jax import lax
from jax.experimental import pallas as pl
from jax.experimental.pallas import tpu as pltpu
from jax.experimental.pallas import tpu_sc as plsc

_B = 1024
_D = 64
_N = 100000
_INV_TEMP = 20.0
_SHIFT = 20.0
_TILE = 5000
_STEPS = _N // _TILE
_LOG2E = 1.4426950408889634
_C = _SHIFT * _LOG2E

_NC = 2
_NS = 16
_NW = _NC * _NS
_BPW = _B // _NW


def _sc_gather(table_wide, idx):
    mesh = plsc.VectorSubcoreMesh(core_axis_name="c", subcore_axis_name="s")

    @functools.partial(
        pl.kernel,
        mesh=mesh,
        out_type=jax.ShapeDtypeStruct((_B, 2 * _D), jnp.float32),
        scratch_types=[
            pltpu.VMEM((_BPW,), jnp.int32),
            pltpu.VMEM((_BPW, 2 * _D), jnp.float32),
            pltpu.SemaphoreType.DMA,
        ],
    )
    def gather_k(table_hbm, idx_hbm, out_hbm, idx_v, rows_v, sem):
        wid = lax.axis_index("s") * _NC + lax.axis_index("c")
        base = wid * _BPW
        pltpu.sync_copy(idx_hbm.at[pl.ds(base, _BPW)], idx_v)
        pltpu.async_copy(table_hbm.at[idx_v], rows_v, sem).wait()
        pltpu.sync_copy(rows_v, out_hbm.at[pl.ds(base, _BPW)])

    return gather_k(table_wide, idx)


def _tc_ce_kernel(x_ref, g_ref, p_ref, f_ref, out_ref, xn_ref, xs_ref, s_ref):
    k = pl.program_id(0)

    @pl.when(k == 0)
    def _():
        x = x_ref[...]
        nrm = jnp.sqrt(jnp.sum(x * x, axis=1, keepdims=True))
        xn = x / jnp.maximum(nrm, 1e-12)
        xn_ref[...] = xn
        xs_ref[...] = (xn * (_INV_TEMP * _LOG2E)).astype(jnp.bfloat16)
        s_ref[...] = jnp.zeros_like(s_ref)

    logits2 = lax.dot_general(
        xs_ref[...], f_ref[...].astype(jnp.bfloat16),
        (((1,), (1,)), ((), ())),
        preferred_element_type=jnp.float32)
    s_ref[...] += jnp.sum(jnp.exp2(logits2 - _C), axis=1, keepdims=True)

    @pl.when(k == _STEPS - 1)
    def _():
        gw = g_ref[...]
        g = jnp.where(p_ref[...] == 1, gw[:, _D:], gw[:, :_D])
        tgt = jnp.sum(xn_ref[...] * g, axis=1, keepdims=True) * _INV_TEMP
        nll = _SHIFT + jnp.log(s_ref[...]) - tgt
        out_ref[0, 0] = jnp.sum(nll) * (1.0 / _B)


def _tc_ce(inputs, gathered_wide, parity, features, interpret=False):
    return pl.pallas_call(
        _tc_ce_kernel,
        grid=(_STEPS,),
        in_specs=[
            pl.BlockSpec((_B, _D), lambda k: (0, 0)),
            pl.BlockSpec((_B, 2 * _D), lambda k: (0, 0)),
            pl.BlockSpec((_B, 1), lambda k: (0, 0)),
            pl.BlockSpec((_TILE, _D), lambda k: (k, 0)),
        ],
        out_specs=pl.BlockSpec(memory_space=pltpu.SMEM),
        out_shape=jax.ShapeDtypeStruct((1, 1), jnp.float32),
        scratch_shapes=[
            pltpu.VMEM((_B, _D), jnp.float32),
            pltpu.VMEM((_B, _D), jnp.bfloat16),
            pltpu.VMEM((_B, 1), jnp.float32),
        ],
        compiler_params=pltpu.CompilerParams(
            dimension_semantics=("arbitrary",)),
        interpret=interpret,
    )(inputs, gathered_wide, parity, features)


def kernel(inputs, targets, features):
    idx = targets.astype(jnp.int32)
    gathered_wide = _sc_gather(features.reshape(_N // 2, 2 * _D), idx // 2)
    parity = (idx % 2).reshape(_B, 1)
    out = _tc_ce(inputs, gathered_wide, parity, features)
    return out[0, 0]


# speedup vs baseline: 1.0033x; 1.0033x over previous
import functools

import jax
import jax.numpy as jnp
from jax import lax
from jax.experimental import pallas as pl
from jax.experimental.pallas import tpu as pltpu
from jax.experimental.pallas import tpu_sc as plsc

_B = 1024
_D = 64
_N = 100000
_INV_TEMP = 20.0
_SHIFT = 20.0
_TILE = 5000
_STEPS = _N // _TILE
_LOG2E = 1.4426950408889634
_C = _SHIFT * _LOG2E

_NC = 2
_NS = 16
_NW = _NC * _NS
_BPW = _B // _NW


def _sc_gather(table_wide, idx):
    mesh = plsc.VectorSubcoreMesh(core_axis_name="c", subcore_axis_name="s")

    @functools.partial(
        pl.kernel,
        mesh=mesh,
        out_type=jax.ShapeDtypeStruct((_B, 2 * _D), jnp.float32),
        scratch_types=[
            pltpu.VMEM((_BPW,), jnp.int32),
            pltpu.VMEM((_BPW, 2 * _D), jnp.float32),
            pltpu.SemaphoreType.DMA,
        ],
    )
    def gather_k(table_hbm, idx_hbm, out_hbm, idx_v, rows_v, sem):
        wid = lax.axis_index("s") * _NC + lax.axis_index("c")
        base = wid * _BPW
        pltpu.sync_copy(idx_hbm.at[pl.ds(base, _BPW)], idx_v)
        idx_v[...] = lax.shift_right_logical(idx_v[...], 1)
        pltpu.async_copy(table_hbm.at[idx_v], rows_v, sem).wait()
        pltpu.sync_copy(rows_v, out_hbm.at[pl.ds(base, _BPW)])

    return gather_k(table_wide, idx)


def _tc_ce_kernel(x_ref, g_ref, t_ref, f_ref, out_ref, xn_ref, xs_ref, s_ref):
    k = pl.program_id(0)

    @pl.when(k == 0)
    def _():
        x = x_ref[...]
        nrm = jnp.sqrt(jnp.sum(x * x, axis=1, keepdims=True))
        xn = x / jnp.maximum(nrm, 1e-12)
        xn_ref[...] = xn
        xs_ref[...] = (xn * (_INV_TEMP * _LOG2E)).astype(jnp.bfloat16)
        s_ref[...] = jnp.zeros_like(s_ref)

    logits2 = lax.dot_general(
        xs_ref[...], f_ref[...].astype(jnp.bfloat16),
        (((1,), (1,)), ((), ())),
        preferred_element_type=jnp.float32)
    s_ref[...] += jnp.sum(jnp.exp2(logits2 - _C), axis=1, keepdims=True)

    @pl.when(k == _STEPS - 1)
    def _():
        gw = g_ref[...]
        p = jnp.reshape(t_ref[...] & 1, (_B, 1))
        g = jnp.where(p == 1, gw[:, _D:], gw[:, :_D])
        tgt = jnp.sum(xn_ref[...] * g, axis=1, keepdims=True) * _INV_TEMP
        nll = _SHIFT + jnp.log(s_ref[...]) - tgt
        out_ref[0, 0] = jnp.sum(nll) * (1.0 / _B)


def _tc_ce(inputs, gathered_wide, targets, features, interpret=False):
    return pl.pallas_call(
        _tc_ce_kernel,
        grid=(_STEPS,),
        in_specs=[
            pl.BlockSpec((_B, _D), lambda k: (0, 0)),
            pl.BlockSpec((_B, 2 * _D), lambda k: (0, 0)),
            pl.BlockSpec((_B,), lambda k: (0,)),
            pl.BlockSpec((_TILE, _D), lambda k: (k, 0)),
        ],
        out_specs=pl.BlockSpec(memory_space=pltpu.SMEM),
        out_shape=jax.ShapeDtypeStruct((1, 1), jnp.float32),
        scratch_shapes=[
            pltpu.VMEM((_B, _D), jnp.float32),
            pltpu.VMEM((_B, _D), jnp.bfloat16),
            pltpu.VMEM((_B, 1), jnp.float32),
        ],
        compiler_params=pltpu.CompilerParams(
            dimension_semantics=("arbitrary",)),
        interpret=interpret,
    )(inputs, gathered_wide, targets, features)


def kernel(inputs, targets, features):
    idx = targets.astype(jnp.int32)
    gathered_wide = _sc_gather(features.reshape(_N // 2, 2 * _D), idx)
    out = _tc_ce(inputs, gathered_wide, idx, features)
    return out[0, 0]
